# baseline (speedup 1.0000x reference)
import functools

import jax
import jax.numpy as jnp
from jax import lax
from jax.experimental import pallas as pl
from jax.experimental.pallas import tpu as pltpu
from jax.experimental.pallas import tpu_sc as plsc

N = 10000
E = 320000
C = 128
H = 8

NC = 2
NS = 16
HPC = H // NC
EPT = E // NS
CHUNK = 32
NCHUNK = EPT // CHUNK
NBUF = 3
NPAD = 10240
DROWS = NPAD // C
RPT = NPAD // NS
OB = 16

BLK = 1000
NB = N // BLK


def _prologue_body(x_ref, w_ref, asrc_ref, adst_ref, hf_ref, as_ref, ad_ref):
    xb = x_ref[...]
    wb = w_ref[0]
    hm = jnp.dot(xb, wb, preferred_element_type=jnp.float32)
    hf_ref[0] = hm
    a_s = asrc_ref[0, 0]
    a_d = adst_ref[0, 0]
    as_ref[0, 0, 0, :] = jnp.sum(hm * a_s[None, :], axis=1)
    ad_ref[0, 0, 0, :] = jnp.sum(hm * a_d[None, :], axis=1)


def _prologue(x, W, att_src, att_dst):
    return pl.pallas_call(
        _prologue_body,
        grid=(H, NB),
        in_specs=[
            pl.BlockSpec((BLK, C), lambda h, nb: (nb, 0)),
            pl.BlockSpec((1, C, C), lambda h, nb: (h, 0, 0)),
            pl.BlockSpec((1, 1, C), lambda h, nb: (h, 0, 0)),
            pl.BlockSpec((1, 1, C), lambda h, nb: (h, 0, 0)),
        ],
        out_specs=[
            pl.BlockSpec((1, BLK, C), lambda h, nb: (h, nb, 0)),
            pl.BlockSpec((1, 1, 1, BLK), lambda h, nb: (h, nb, 0, 0)),
            pl.BlockSpec((1, 1, 1, BLK), lambda h, nb: (h, nb, 0, 0)),
        ],
        out_shape=[
            jax.ShapeDtypeStruct((H, N, C), jnp.float32),
            jax.ShapeDtypeStruct((H, NB, 1, BLK), jnp.float32),
            jax.ShapeDtypeStruct((H, NB, 1, BLK), jnp.float32),
        ],
        compiler_params=pltpu.CompilerParams(
            dimension_semantics=("parallel", "parallel")),
    )(x, W, att_src.reshape(H, 1, C), att_dst.reshape(H, 1, C))


def _sc_body(hfeat_hbm, asrc_hbm, adst_hbm, bias_hbm, src_hbm, dst_hbm,
             out_hbm,
             srci_v, dsti_v, dstl_v, srcabs_v, w_v, rows_v, asrc_v, adst_v,
             den_v,
             biasl_v, accl_v, rowidx_v, acc_sh, den_sh,
             isem0, isem1, isem2, jsem0, jsem1, jsem2,
             gsem0, gsem1, gsem2, ssem0, ssem1, ssem2):
    isems = (isem0, isem1, isem2)
    jsems = (jsem0, jsem1, jsem2)
    gsems = (gsem0, gsem1, gsem2)
    ssems = (ssem0, ssem1, ssem2)
    cid = lax.axis_index("c")
    sid = lax.axis_index("s")
    ebase0 = sid * EPT

    for g in range(DROWS // 16):
        rowidx_v[pl.ds(g * 16, 16)] = (
            lax.iota(jnp.int32, 16) + g * 16)

    def _head(hloc, carry):
        h = cid * HPC + hloc
        plsc.subcore_barrier()
        def _zd(i, c2):
            for j in range(C // 16):
                den_v[i, pl.ds(j * 16, 16)] = jnp.zeros((16,), jnp.float32)
            return c2
        lax.fori_loop(0, DROWS, _zd, 0)
        def _za(i, c2):
            for j in range(C // 16):
                accl_v[i, pl.ds(j * 16, 16)] = jnp.zeros((16,), jnp.float32)
            return c2
        lax.fori_loop(0, OB, _za, 0)
        for z in range(RPT // OB):
            pltpu.sync_copy(accl_v, acc_sh.at[pl.ds(sid * RPT + z * OB, OB)])
        @pl.when(sid == 0)
        def _zero_den_sh():
            pltpu.sync_copy(den_v, den_sh.at[rowidx_v])
        pltpu.sync_copy(asrc_hbm.at[h], asrc_v)
        pltpu.sync_copy(adst_hbm.at[h], adst_v)
        pltpu.sync_copy(bias_hbm.at[h], biasl_v)
        plsc.subcore_barrier()
        hN = h * N

        def _fire_idx(c, b):
            eb = ebase0 + c * CHUNK
            pltpu.async_copy(src_hbm.at[pl.ds(eb, CHUNK)], srci_v.at[b],
                             isems[b])
            pltpu.async_copy(dst_hbm.at[pl.ds(eb, CHUNK)], dsti_v.at[b],
                             jsems[b])

        def _mid(c, b):
            eb = ebase0 + c * CHUNK
            pltpu.make_async_copy(src_hbm.at[pl.ds(eb, CHUNK)],
                                  srci_v.at[b], isems[b]).wait()
            pltpu.make_async_copy(dst_hbm.at[pl.ds(eb, CHUNK)],
                                  dsti_v.at[b], jsems[b]).wait()
            for g in range(CHUNK // 16):
                sl = pl.ds(g * 16, 16)
                s16 = srci_v[b, sl]
                d16 = dsti_v[b, sl]
                srcabs_v[b, sl] = s16 + hN
                dstl_v[b, sl] = d16
                e = (plsc.load_gather(asrc_v, [s16])
                     + plsc.load_gather(adst_v, [d16]))
                e = jnp.where(e > 0, e, 0.2 * e)
                w = jnp.exp(e)
                w_v[b, sl] = w
                plsc.addupdate_scatter(
                    den_v,
                    [lax.shift_right_logical(d16, 7),
                     lax.bitwise_and(d16, 127)], w)
            pltpu.async_copy(hfeat_hbm.at[srcabs_v.at[b]], rows_v.at[b],
                             gsems[b])

        def _drain_scatter(b):
            pltpu.make_async_copy(rows_v.at[b], acc_sh.at[dstl_v.at[b]],
                                  ssems[b]).wait()

        def _proc(c, b):
            pltpu.make_async_copy(hfeat_hbm.at[srcabs_v.at[b]],
                                  rows_v.at[b], gsems[b]).wait()
            def _scale16(i0, carry3):
                for u in range(16):
                    i = i0 * 16 + u
                    wb = plsc.load_gather(
                        w_v, [jnp.full((16,), b, jnp.int32),
                              jnp.full((16,), i, jnp.int32)])
                    for j in range(C // 16):
                        sl = pl.ds(j * 16, 16)
                        rows_v[b, i, sl] = rows_v[b, i, sl] * wb
                return carry3
            lax.fori_loop(0, CHUNK // 16, _scale16, 0)
            pltpu.async_copy(rows_v.at[b], acc_sh.at[dstl_v.at[b]],
                             ssems[b], add=True)

        def _body(c, b):
            _proc(c, b)
            bn = (b + 2) % NBUF
            @pl.when(c >= 1)
            def _():
                _drain_scatter(bn)
            @pl.when(c + 2 <= NCHUNK - 1)
            def _():
                _mid(c + 2, bn)
            @pl.when(c + 3 <= NCHUNK - 1)
            def _():
                _fire_idx(c + 3, b)

        _fire_idx(jnp.int32(0), 0)
        _fire_idx(jnp.int32(1), 1)
        _fire_idx(jnp.int32(2), 2)
        _mid(jnp.int32(0), 0)
        _mid(jnp.int32(1), 1)

        def _super(k, carry2):
            for b in range(NBUF):
                _body(k * NBUF + b, b)
            return carry2
        lax.fori_loop(0, NCHUNK // NBUF, _super, 0)
        for c in range(NCHUNK - NCHUNK % NBUF, NCHUNK):
            _body(jnp.int32(c), c % NBUF)
        _drain_scatter((NCHUNK - 1) % NBUF)

        pltpu.sync_copy(den_v, den_sh.at[rowidx_v], add=True)
        plsc.subcore_barrier()

        pltpu.sync_copy(den_sh, den_v)
        for ob in range(RPT // OB):
            r0 = sid * RPT + ob * OB
            pltpu.sync_copy(acc_sh.at[pl.ds(r0, OB)], accl_v)

            def _norm(i, c3):
                gr = r0 + i
                dv = plsc.load_gather(
                    den_v,
                    [jnp.full((16,), lax.shift_right_logical(gr, 7),
                              jnp.int32),
                     jnp.full((16,), lax.bitwise_and(gr, 127), jnp.int32)])
                recip = 1.0 / (dv + 1e-16)
                for j in range(C // 16):
                    sl = pl.ds(j * 16, 16)
                    accl_v[i, sl] = accl_v[i, sl] * recip + biasl_v[sl]
                return c3
            lax.fori_loop(0, OB, _norm, 0)
            pltpu.sync_copy(accl_v, out_hbm.at[h, pl.ds(r0, OB)])
        return carry
    lax.fori_loop(0, HPC, _head, 0)


def _sc_stage(hfeat2, asrc, adst, bias, src, dst):
    mesh = plsc.VectorSubcoreMesh(core_axis_name="c", subcore_axis_name="s")
    kern = functools.partial(
        pl.kernel,
        out_type=jax.ShapeDtypeStruct((H, NPAD, C), jnp.float32),
        mesh=mesh,
        scratch_types=[
            pltpu.VMEM((NBUF, CHUNK), jnp.int32),
            pltpu.VMEM((NBUF, CHUNK), jnp.int32),
            pltpu.VMEM((NBUF, CHUNK), jnp.int32),
            pltpu.VMEM((NBUF, CHUNK), jnp.int32),
            pltpu.VMEM((NBUF, CHUNK), jnp.float32),
            pltpu.VMEM((NBUF, CHUNK, C), jnp.float32),
            pltpu.VMEM((N,), jnp.float32),
            pltpu.VMEM((N,), jnp.float32),
            pltpu.VMEM((DROWS, C), jnp.float32),
            pltpu.VMEM((C,), jnp.float32),
            pltpu.VMEM((OB, C), jnp.float32),
            pltpu.VMEM((DROWS,), jnp.int32),
            pltpu.VMEM_SHARED((NPAD, C), jnp.float32),
            pltpu.VMEM_SHARED((DROWS, C), jnp.float32),
        ] + [pltpu.SemaphoreType.DMA] * 12,
        compiler_params=pltpu.CompilerParams(needs_layout_passes=False),
    )(_sc_body)
    return kern(hfeat2, asrc, adst, bias, src, dst)


def kernel(x, edge_index, W, att_src, att_dst, bias):
    edge_index = edge_index.astype(jnp.int32)
    src, dst = edge_index[0], edge_index[1]
    hfeat, asrc4, adst4 = _prologue(x, W, att_src, att_dst)
    out = _sc_stage(hfeat.reshape(H * N, C),
                    asrc4.reshape(H, N), adst4.reshape(H, N), bias, src, dst)
    return tuple(out[h, :N] for h in range(H))


# speedup vs baseline: 1.2664x; 1.2664x over previous
import functools

import jax
import jax.numpy as jnp
from jax import lax
from jax.experimental import pallas as pl
from jax.experimental.pallas import tpu as pltpu
from jax.experimental.pallas import tpu_sc as plsc

N = 10000
E = 320000
C = 128
H = 8

NC = 2
NS = 16
HPC = H // NC
NPAD = 10240
DROWS = NPAD // C
RPT = NPAD // NS
OB = 16

TPH = NS // HPC
EPQ = E // TPH
CHL = 640
NCHL = EPQ // CHL
LBUF = 2

EPT = E // NS
CHUNK = 80
NCHUNK = EPT // CHUNK
NBUF = 3
DSL = RPT // C

BLK = 1000
NB = N // BLK


def _prologue_body(x_ref, w_ref, asrc_ref, adst_ref, hf_ref, as_ref, ad_ref):
    xb = x_ref[...]
    wb = w_ref[0]
    hm = jnp.dot(xb, wb, preferred_element_type=jnp.float32)
    hf_ref[0] = hm
    a_s = asrc_ref[0, 0]
    a_d = adst_ref[0, 0]
    as_ref[0, 0, 0, :] = jnp.sum(hm * a_s[None, :], axis=1)
    ad_ref[0, 0, 0, :] = jnp.sum(hm * a_d[None, :], axis=1)


def _prologue(x, W, att_src, att_dst):
    return pl.pallas_call(
        _prologue_body,
        grid=(H, NB),
        in_specs=[
            pl.BlockSpec((BLK, C), lambda h, nb: (nb, 0)),
            pl.BlockSpec((1, C, C), lambda h, nb: (h, 0, 0)),
            pl.BlockSpec((1, 1, C), lambda h, nb: (h, 0, 0)),
            pl.BlockSpec((1, 1, C), lambda h, nb: (h, 0, 0)),
        ],
        out_specs=[
            pl.BlockSpec((1, BLK, C), lambda h, nb: (h, nb, 0)),
            pl.BlockSpec((1, 1, 1, BLK), lambda h, nb: (h, nb, 0, 0)),
            pl.BlockSpec((1, 1, 1, BLK), lambda h, nb: (h, nb, 0, 0)),
        ],
        out_shape=[
            jax.ShapeDtypeStruct((H, N, C), jnp.float32),
            jax.ShapeDtypeStruct((H, NB, 1, BLK), jnp.float32),
            jax.ShapeDtypeStruct((H, NB, 1, BLK), jnp.float32),
        ],
        compiler_params=pltpu.CompilerParams(
            dimension_semantics=("parallel", "parallel")),
    )(x, W, att_src.reshape(H, 1, C), att_dst.reshape(H, 1, C))


def _logit_body(asrc_hbm, adst_hbm, src_hbm, dst_hbm,
                w_hbm, den_hbm,
                srci_v, dsti_v, wbuf_v, asrc_v, adst_v, den_v, rowidx_v,
                den_sh,
                isem0, isem1, jsem0, jsem1, wsem0, wsem1):
    isems = (isem0, isem1)
    jsems = (jsem0, jsem1)
    wsems = (wsem0, wsem1)
    cid = lax.axis_index("c")
    sid = lax.axis_index("s")
    hloc = sid // TPH
    h = cid * HPC + hloc
    e0 = (sid % TPH) * EPQ
    wbase = h * E + e0

    for g in range(DROWS // 16):
        rowidx_v[pl.ds(g * 16, 16)] = lax.iota(jnp.int32, 16) + g * 16

    def _zd(i, c2):
        for j in range(C // 16):
            den_v[i, pl.ds(j * 16, 16)] = jnp.zeros((16,), jnp.float32)
        return c2
    lax.fori_loop(0, DROWS, _zd, 0)
    @pl.when(sid % TPH == 0)
    def _zds():
        pltpu.sync_copy(den_v, den_sh.at[hloc].at[rowidx_v])
    pltpu.sync_copy(asrc_hbm.at[h], asrc_v)
    pltpu.sync_copy(adst_hbm.at[h], adst_v)
    plsc.subcore_barrier()

    def _fire_idx(c, b):
        eb = e0 + c * CHL
        pltpu.async_copy(src_hbm.at[pl.ds(eb, CHL)], srci_v.at[b], isems[b])
        pltpu.async_copy(dst_hbm.at[pl.ds(eb, CHL)], dsti_v.at[b], jsems[b])

    def _body(c, b):
        eb = e0 + c * CHL
        pltpu.make_async_copy(src_hbm.at[pl.ds(eb, CHL)],
                              srci_v.at[b], isems[b]).wait()
        pltpu.make_async_copy(dst_hbm.at[pl.ds(eb, CHL)],
                              dsti_v.at[b], jsems[b]).wait()
        @pl.when(c >= LBUF)
        def _():
            pltpu.make_async_copy(
                wbuf_v.at[b],
                w_hbm.at[pl.ds(wbase + (c - LBUF) * CHL, CHL)],
                wsems[b]).wait()

        def _grp(g, c3):
            sl = pl.ds(g * 16, 16)
            s16 = srci_v[b, sl]
            d16 = dsti_v[b, sl]
            e = (plsc.load_gather(asrc_v, [s16])
                 + plsc.load_gather(adst_v, [d16]))
            e = jnp.where(e > 0, e, 0.2 * e)
            w = jnp.exp(e)
            wbuf_v[b, sl] = w
            plsc.addupdate_scatter(
                den_v,
                [lax.shift_right_logical(d16, 7),
                 lax.bitwise_and(d16, 127)], w)
            return c3
        lax.fori_loop(0, CHL // 16, _grp, 0)
        pltpu.async_copy(wbuf_v.at[b],
                         w_hbm.at[pl.ds(wbase + c * CHL, CHL)], wsems[b])
        @pl.when(c + LBUF <= NCHL - 1)
        def _():
            _fire_idx(c + LBUF, b)

    _fire_idx(jnp.int32(0), 0)
    _fire_idx(jnp.int32(1), 1)

    def _super(k, carry):
        for b in range(LBUF):
            _body(k * LBUF + b, b)
        return carry
    lax.fori_loop(0, NCHL // LBUF, _super, 0)
    for c in range(NCHL - NCHL % LBUF, NCHL):
        _body(jnp.int32(c), c % LBUF)
    for c in range(NCHL - LBUF, NCHL):
        pltpu.make_async_copy(
            wbuf_v.at[c % LBUF],
            w_hbm.at[pl.ds(wbase + c * CHL, CHL)],
            wsems[c % LBUF]).wait()

    pltpu.sync_copy(den_v, den_sh.at[hloc].at[rowidx_v], add=True)
    plsc.subcore_barrier()
    @pl.when(sid % TPH == 0)
    def _wden():
        pltpu.sync_copy(den_sh.at[hloc], den_hbm.at[h])


def _logit_stage(asrc, adst, src, dst):
    mesh = plsc.VectorSubcoreMesh(core_axis_name="c", subcore_axis_name="s")
    kern = functools.partial(
        pl.kernel,
        out_type=[
            jax.ShapeDtypeStruct((H * E,), jnp.float32),
            jax.ShapeDtypeStruct((H, DROWS, C), jnp.float32),
        ],
        mesh=mesh,
        scratch_types=[
            pltpu.VMEM((LBUF, CHL), jnp.int32),
            pltpu.VMEM((LBUF, CHL), jnp.int32),
            pltpu.VMEM((LBUF, CHL), jnp.float32),
            pltpu.VMEM((N,), jnp.float32),
            pltpu.VMEM((N,), jnp.float32),
            pltpu.VMEM((DROWS, C), jnp.float32),
            pltpu.VMEM((DROWS,), jnp.int32),
            pltpu.VMEM_SHARED((HPC, DROWS, C), jnp.float32),
        ] + [pltpu.SemaphoreType.DMA] * 6,
        compiler_params=pltpu.CompilerParams(needs_layout_passes=False),
    )(_logit_body)
    return kern(asrc, adst, src, dst)


def _edge_body(hfeat_hbm, w_hbm, den_hbm, bias_hbm, src_hbm, dst_hbm,
               out_hbm,
               srci_v, dsti_v, dstl_v, srcabs_v, wl_v, rows_v, denl_v,
               biasl_v, accl_v, acc_sh,
               isem0, isem1, isem2, jsem0, jsem1, jsem2,
               wsem0, wsem1, wsem2, gsem0, gsem1, gsem2,
               ssem0, ssem1, ssem2):
    isems = (isem0, isem1, isem2)
    jsems = (jsem0, jsem1, jsem2)
    wsems = (wsem0, wsem1, wsem2)
    gsems = (gsem0, gsem1, gsem2)
    ssems = (ssem0, ssem1, ssem2)
    cid = lax.axis_index("c")
    sid = lax.axis_index("s")
    ebase0 = sid * EPT

    def _head(hloc, carry):
        h = cid * HPC + hloc
        plsc.subcore_barrier()
        def _za(i, c2):
            for j in range(C // 16):
                accl_v[i, pl.ds(j * 16, 16)] = jnp.zeros((16,), jnp.float32)
            return c2
        lax.fori_loop(0, OB, _za, 0)
        for z in range(RPT // OB):
            pltpu.sync_copy(accl_v, acc_sh.at[pl.ds(sid * RPT + z * OB, OB)])
        pltpu.sync_copy(den_hbm.at[pl.ds(h * NPAD + sid * RPT, RPT)],
                        denl_v)
        pltpu.sync_copy(bias_hbm.at[h], biasl_v)
        plsc.subcore_barrier()
        hN = h * N
        wb0 = h * E + ebase0

        def _fire_idx(c, b):
            eb = ebase0 + c * CHUNK
            pltpu.async_copy(src_hbm.at[pl.ds(eb, CHUNK)], srci_v.at[b],
                             isems[b])
            pltpu.async_copy(dst_hbm.at[pl.ds(eb, CHUNK)], dsti_v.at[b],
                             jsems[b])
            pltpu.async_copy(w_hbm.at[pl.ds(wb0 + c * CHUNK, CHUNK)],
                             wl_v.at[b], wsems[b])

        def _mid(c, b):
            eb = ebase0 + c * CHUNK
            pltpu.make_async_copy(src_hbm.at[pl.ds(eb, CHUNK)],
                                  srci_v.at[b], isems[b]).wait()
            pltpu.make_async_copy(dst_hbm.at[pl.ds(eb, CHUNK)],
                                  dsti_v.at[b], jsems[b]).wait()
            for g in range(CHUNK // 16):
                sl = pl.ds(g * 16, 16)
                srcabs_v[b, sl] = srci_v[b, sl] + hN
                dstl_v[b, sl] = dsti_v[b, sl]
            pltpu.async_copy(hfeat_hbm.at[srcabs_v.at[b]], rows_v.at[b],
                             gsems[b])

        def _drain_scatter(b):
            pltpu.make_async_copy(rows_v.at[b], acc_sh.at[dstl_v.at[b]],
                                  ssems[b]).wait()

        def _proc(c, b):
            pltpu.make_async_copy(hfeat_hbm.at[srcabs_v.at[b]],
                                  rows_v.at[b], gsems[b]).wait()
            pltpu.make_async_copy(
                w_hbm.at[pl.ds(wb0 + c * CHUNK, CHUNK)],
                wl_v.at[b], wsems[b]).wait()
            def _scale8(i0, carry3):
                for u in range(8):
                    i = i0 * 8 + u
                    wv = plsc.load_gather(
                        wl_v, [jnp.full((16,), b, jnp.int32),
                               jnp.full((16,), i, jnp.int32)])
                    for j in range(C // 16):
                        sl = pl.ds(j * 16, 16)
                        rows_v[b, i, sl] = rows_v[b, i, sl] * wv
                return carry3
            lax.fori_loop(0, CHUNK // 8, _scale8, 0)
            pltpu.async_copy(rows_v.at[b], acc_sh.at[dstl_v.at[b]],
                             ssems[b], add=True)

        def _body(c, b):
            _proc(c, b)
            bn = (b + 2) % NBUF
            @pl.when(c >= 1)
            def _():
                _drain_scatter(bn)
            @pl.when(c + 2 <= NCHUNK - 1)
            def _():
                _mid(c + 2, bn)
            @pl.when(c + 3 <= NCHUNK - 1)
            def _():
                _fire_idx(c + 3, b)

        _fire_idx(jnp.int32(0), 0)
        _fire_idx(jnp.int32(1), 1)
        _fire_idx(jnp.int32(2), 2)
        _mid(jnp.int32(0), 0)
        _mid(jnp.int32(1), 1)

        def _super(k, carry2):
            for b in range(NBUF):
                _body(k * NBUF + b, b)
            return carry2
        lax.fori_loop(0, NCHUNK // NBUF, _super, 0)
        for c in range(NCHUNK - NCHUNK % NBUF, NCHUNK):
            _body(jnp.int32(c), c % NBUF)
        _drain_scatter((NCHUNK - 1) % NBUF)

        plsc.subcore_barrier()

        for ob in range(RPT // OB):
            r0 = sid * RPT + ob * OB
            pltpu.sync_copy(acc_sh.at[pl.ds(r0, OB)], accl_v)

            def _norm(i, c3):
                lr = ob * OB + i
                dv = plsc.load_gather(denl_v,
                                      [jnp.full((16,), lr, jnp.int32)])
                recip = 1.0 / (dv + 1e-16)
                for j in range(C // 16):
                    sl = pl.ds(j * 16, 16)
                    accl_v[i, sl] = accl_v[i, sl] * recip + biasl_v[sl]
                return c3
            lax.fori_loop(0, OB, _norm, 0)
            pltpu.sync_copy(accl_v, out_hbm.at[h, pl.ds(r0, OB)])
        return carry
    lax.fori_loop(0, HPC, _head, 0)


def _edge_stage(hfeat2, w_all, den_all, bias, src, dst):
    mesh = plsc.VectorSubcoreMesh(core_axis_name="c", subcore_axis_name="s")
    kern = functools.partial(
        pl.kernel,
        out_type=jax.ShapeDtypeStruct((H, NPAD, C), jnp.float32),
        mesh=mesh,
        scratch_types=[
            pltpu.VMEM((NBUF, CHUNK), jnp.int32),
            pltpu.VMEM((NBUF, CHUNK), jnp.int32),
            pltpu.VMEM((NBUF, CHUNK), jnp.int32),
            pltpu.VMEM((NBUF, CHUNK), jnp.int32),
            pltpu.VMEM((NBUF, CHUNK), jnp.float32),
            pltpu.VMEM((NBUF, CHUNK, C), jnp.float32),
            pltpu.VMEM((RPT,), jnp.float32),
            pltpu.VMEM((C,), jnp.float32),
            pltpu.VMEM((OB, C), jnp.float32),
            pltpu.VMEM_SHARED((NPAD, C), jnp.float32),
        ] + [pltpu.SemaphoreType.DMA] * 15,
        compiler_params=pltpu.CompilerParams(needs_layout_passes=False),
    )(_edge_body)
    return kern(hfeat2, w_all, den_all, bias, src, dst)


def kernel(x, edge_index, W, att_src, att_dst, bias):
    edge_index = edge_index.astype(jnp.int32)
    src, dst = edge_index[0], edge_index[1]
    hfeat, asrc4, adst4 = _prologue(x, W, att_src, att_dst)
    w_all, den_all = _logit_stage(
        asrc4.reshape(H, N), adst4.reshape(H, N), src, dst)
    out = _edge_stage(hfeat.reshape(H * N, C), w_all,
                      den_all.reshape(H * NPAD), bias, src, dst)
    return tuple(out[h, :N] for h in range(H))
